# Initial kernel scaffold; baseline (speedup 1.0000x reference)
import jax
import jax.numpy as jnp
from jax.experimental import pallas as pl


def kernel(emb, idx):
    raise NotImplementedError("write your pallas kernel here")


# speedup vs baseline: 2.4892x; 2.4892x over previous
import functools

import jax
import jax.numpy as jnp
from jax import lax
from jax.experimental import pallas as pl
from jax.experimental.pallas import tpu as pltpu
from jax.experimental.pallas import tpu_sc as plsc

_NC = 2
_NS = 16
_NW = _NC * _NS

_CHUNK = 2048
_STREAM = 128


def _gather_kernel(B, D, table_hbm, idx_hbm, out_hbm, idx_v, rows_v, sem):
    b_per_w = B // _NW
    n_chunks = b_per_w // _CHUNK
    wid = lax.axis_index("s") * _NC + lax.axis_index("c")
    base = wid * b_per_w

    def body(c, carry):
        off = base + c * _CHUNK
        pltpu.sync_copy(idx_hbm.at[pl.ds(off, _CHUNK)], idx_v)
        copies = []
        for j in range(_CHUNK // _STREAM):
            cp = pltpu.async_copy(
                table_hbm.at[idx_v.at[pl.ds(j * _STREAM, _STREAM)]],
                rows_v.at[pl.ds(j * _STREAM, _STREAM)],
                sem,
            )
            copies.append(cp)
        for cp in copies:
            cp.wait()
        pltpu.sync_copy(rows_v, out_hbm.at[pl.ds(off, _CHUNK)])
        return carry

    lax.fori_loop(0, n_chunks, body, 0)


def kernel(emb, idx):
    V, D = emb.shape
    B0, B1 = idx.shape
    B = B0 * B1
    idx_flat = idx.reshape(B)

    mesh = plsc.VectorSubcoreMesh(core_axis_name="c", subcore_axis_name="s")
    gather = functools.partial(
        pl.kernel,
        mesh=mesh,
        out_type=jax.ShapeDtypeStruct((B, D), jnp.float32),
        scratch_types=[
            pltpu.VMEM((_CHUNK,), jnp.int32),
            pltpu.VMEM((_CHUNK, D), jnp.float32),
            pltpu.SemaphoreType.DMA,
        ],
        compiler_params=pltpu.CompilerParams(use_tc_tiling_on_sc=False),
    )(functools.partial(_gather_kernel, B, D))

    out = gather(emb, idx_flat)
    return out.reshape(B0, B1, D)


# speedup vs baseline: 2.5331x; 1.0176x over previous
import functools

import jax
import jax.numpy as jnp
from jax import lax
from jax.experimental import pallas as pl
from jax.experimental.pallas import tpu as pltpu
from jax.experimental.pallas import tpu_sc as plsc

_NC = 2
_NS = 16
_NW = _NC * _NS

_CHUNK = 2048
_STREAM = 128


def _gather_kernel(B, D, table_hbm, idx_hbm, out_hbm,
                   idx_v, rows_v, sem_i0, sem_i1, sem_g, sem_s0, sem_s1):
    b_per_w = B // _NW
    n_chunks = b_per_w // _CHUNK
    n_pairs = n_chunks // 2
    wid = lax.axis_index("s") * _NC + lax.axis_index("c")
    base = wid * b_per_w

    pltpu.async_copy(idx_hbm.at[pl.ds(base, _CHUNK)], idx_v.at[0], sem_i0)

    def pair_body(p, carry):
        for b in (0, 1):
            c = 2 * p + b
            sem_i = (sem_i0, sem_i1)[b]
            sem_s = (sem_s0, sem_s1)[b]
            off = base + c * _CHUNK

            @pl.when(p >= 1)
            def _():
                pltpu.make_async_copy(
                    rows_v.at[b], out_hbm.at[pl.ds(0, _CHUNK)], sem_s
                ).wait()

            pltpu.make_async_copy(
                idx_hbm.at[pl.ds(0, _CHUNK)], idx_v.at[b], sem_i
            ).wait()

            def _prefetch():
                pltpu.async_copy(
                    idx_hbm.at[pl.ds(off + _CHUNK, _CHUNK)],
                    idx_v.at[1 - b],
                    (sem_i1, sem_i0)[b],
                )
            if b == 0:
                _prefetch()
            else:
                pl.when(p < n_pairs - 1)(_prefetch)

            copies = []
            for j in range(_CHUNK // _STREAM):
                cp = pltpu.async_copy(
                    table_hbm.at[idx_v.at[b, pl.ds(j * _STREAM, _STREAM)]],
                    rows_v.at[b, pl.ds(j * _STREAM, _STREAM)],
                    sem_g,
                )
                copies.append(cp)
            for cp in copies:
                cp.wait()

            pltpu.async_copy(rows_v.at[b], out_hbm.at[pl.ds(off, _CHUNK)], sem_s)
        return carry

    lax.fori_loop(0, n_pairs, pair_body, 0)

    pltpu.make_async_copy(rows_v.at[0], out_hbm.at[pl.ds(0, _CHUNK)], sem_s0).wait()
    pltpu.make_async_copy(rows_v.at[1], out_hbm.at[pl.ds(0, _CHUNK)], sem_s1).wait()


def kernel(emb, idx):
    V, D = emb.shape
    B0, B1 = idx.shape
    B = B0 * B1
    idx_flat = idx.reshape(B)

    mesh = plsc.VectorSubcoreMesh(core_axis_name="c", subcore_axis_name="s")
    gather = functools.partial(
        pl.kernel,
        mesh=mesh,
        out_type=jax.ShapeDtypeStruct((B, D), jnp.float32),
        scratch_types=[
            pltpu.VMEM((2, _CHUNK), jnp.int32),
            pltpu.VMEM((2, _CHUNK, D), jnp.float32),
            pltpu.SemaphoreType.DMA,
            pltpu.SemaphoreType.DMA,
            pltpu.SemaphoreType.DMA,
            pltpu.SemaphoreType.DMA,
            pltpu.SemaphoreType.DMA,
        ],
        compiler_params=pltpu.CompilerParams(use_tc_tiling_on_sc=False),
    )(functools.partial(_gather_kernel, B, D))

    out = gather(emb, idx_flat)
    return out.reshape(B0, B1, D)


# speedup vs baseline: 2.5335x; 1.0002x over previous
import functools

import jax
import jax.numpy as jnp
from jax import lax
from jax.experimental import pallas as pl
from jax.experimental.pallas import tpu as pltpu
from jax.experimental.pallas import tpu_sc as plsc

_NC = 2
_NS = 16
_NW = _NC * _NS

_CHUNK = 2048
_STREAM = 512


def _gather_kernel(B, D, table_hbm, idx_hbm, out_hbm,
                   idx_v, rows_v, sem_i0, sem_i1, sem_g, sem_s0, sem_s1):
    b_per_w = B // _NW
    n_chunks = b_per_w // _CHUNK
    n_pairs = n_chunks // 2
    wid = lax.axis_index("s") * _NC + lax.axis_index("c")
    base = wid * b_per_w

    pltpu.async_copy(idx_hbm.at[pl.ds(base, _CHUNK)], idx_v.at[0], sem_i0)

    def pair_body(p, carry):
        for b in (0, 1):
            c = 2 * p + b
            sem_i = (sem_i0, sem_i1)[b]
            sem_s = (sem_s0, sem_s1)[b]
            off = base + c * _CHUNK

            @pl.when(p >= 1)
            def _():
                pltpu.make_async_copy(
                    rows_v.at[b], out_hbm.at[pl.ds(0, _CHUNK)], sem_s
                ).wait()

            pltpu.make_async_copy(
                idx_hbm.at[pl.ds(0, _CHUNK)], idx_v.at[b], sem_i
            ).wait()

            def _prefetch():
                pltpu.async_copy(
                    idx_hbm.at[pl.ds(off + _CHUNK, _CHUNK)],
                    idx_v.at[1 - b],
                    (sem_i1, sem_i0)[b],
                )
            if b == 0:
                _prefetch()
            else:
                pl.when(p < n_pairs - 1)(_prefetch)

            copies = []
            for j in range(_CHUNK // _STREAM):
                cp = pltpu.async_copy(
                    table_hbm.at[idx_v.at[b, pl.ds(j * _STREAM, _STREAM)]],
                    rows_v.at[b, pl.ds(j * _STREAM, _STREAM)],
                    sem_g,
                )
                copies.append(cp)
            for cp in copies:
                cp.wait()

            pltpu.async_copy(rows_v.at[b], out_hbm.at[pl.ds(off, _CHUNK)], sem_s)
        return carry

    lax.fori_loop(0, n_pairs, pair_body, 0)

    pltpu.make_async_copy(rows_v.at[0], out_hbm.at[pl.ds(0, _CHUNK)], sem_s0).wait()
    pltpu.make_async_copy(rows_v.at[1], out_hbm.at[pl.ds(0, _CHUNK)], sem_s1).wait()


def kernel(emb, idx):
    V, D = emb.shape
    B0, B1 = idx.shape
    B = B0 * B1
    idx_flat = idx.reshape(B)

    mesh = plsc.VectorSubcoreMesh(core_axis_name="c", subcore_axis_name="s")
    gather = functools.partial(
        pl.kernel,
        mesh=mesh,
        out_type=jax.ShapeDtypeStruct((B, D), jnp.float32),
        scratch_types=[
            pltpu.VMEM((2, _CHUNK), jnp.int32),
            pltpu.VMEM((2, _CHUNK, D), jnp.float32),
            pltpu.SemaphoreType.DMA,
            pltpu.SemaphoreType.DMA,
            pltpu.SemaphoreType.DMA,
            pltpu.SemaphoreType.DMA,
            pltpu.SemaphoreType.DMA,
        ],
        compiler_params=pltpu.CompilerParams(use_tc_tiling_on_sc=False),
    )(functools.partial(_gather_kernel, B, D))

    out = gather(emb, idx_flat)
    return out.reshape(B0, B1, D)
